# speedup vs baseline: 2.7406x; 2.7406x over previous
import jax
import jax.numpy as jnp
from jax import lax
from jax.experimental import pallas as pl

HEAD_DIM = 64
NUM_EMB = 257
K_PAD = 264
M_BLOCK = 8192
CHUNK = 2048


def _gather_kernel(idx_ref, emb_ref, out_ref):
    emb = emb_ref[...]
    for c in range(M_BLOCK // CHUNK):
        sl = pl.ds(c * CHUNK, CHUNK)
        idx = idx_ref[sl, :]
        iota = lax.broadcasted_iota(jnp.int32, (CHUNK, K_PAD), 1)
        onehot = jnp.where(idx == iota, 1.0, 0.0).astype(jnp.float32)
        out_ref[sl, :] = lax.dot_general(
            onehot, emb, (((1,), (0,)), ((), ())),
            preferred_element_type=jnp.float32)


def kernel(input, embeddings):
    n = input.shape[0] * input.shape[1]
    idx2 = input.reshape(n, 1).astype(jnp.int32)
    embp = jnp.zeros((K_PAD, HEAD_DIM), jnp.float32).at[:NUM_EMB].set(embeddings)
    out = pl.pallas_call(
        _gather_kernel,
        grid=(n // M_BLOCK,),
        in_specs=[
            pl.BlockSpec((M_BLOCK, 1), lambda i: (i, 0)),
            pl.BlockSpec((K_PAD, HEAD_DIM), lambda i: (0, 0)),
        ],
        out_specs=pl.BlockSpec((M_BLOCK, HEAD_DIM), lambda i: (i, 0)),
        out_shape=jax.ShapeDtypeStruct((n, HEAD_DIM), jnp.float32),
    )(idx2, embp)
    return out.reshape(input.shape[0], input.shape[1], HEAD_DIM)
